# baseline (speedup 1.0000x reference)
import functools

import jax
import jax.numpy as jnp
from jax import lax
from jax.experimental import pallas as pl
from jax.experimental.pallas import tpu as pltpu
from jax.experimental.pallas import tpu_sc as plsc

N = 10000
NP = 10240
E = 320000
D = 128
DH = 64
CH = 128
NSC = 2
NSUB = 16
EP = 327680
CPT = EP // NSUB // CH
CPT_DEG = EP // (NSC * NSUB) // CH
NBUF = 4
ROWS_PT = NP // NSUB

_mesh = plsc.VectorSubcoreMesh(core_axis_name="c", subcore_axis_name="s")


@functools.partial(
    pl.kernel,
    mesh=_mesh,
    out_type=jax.ShapeDtypeStruct((NSC, NP, 16), jnp.float32),
    scratch_types=[
        pltpu.VMEM((CPT_DEG, CH), jnp.int32),
        pltpu.VMEM((CH, 16), jnp.float32),
        pltpu.VMEM_SHARED((NP, 16), jnp.float32),
        pltpu.SemaphoreType.DMA,
    ],
    compiler_params=pltpu.CompilerParams(use_tc_tiling_on_sc=False),
)
def _deg_kernel(edge_hbm, ones_hbm, zeros_hbm, out_hbm, idx_v, ones_v, acc_sh, sem):
    c = lax.axis_index("c")
    s = lax.axis_index("s")
    w = c * NSUB + s
    base = s * ROWS_PT
    pltpu.sync_copy(edge_hbm.at[1, pl.ds(w * CPT_DEG, CPT_DEG)], idx_v)
    pltpu.sync_copy(ones_hbm, ones_v)
    pltpu.sync_copy(zeros_hbm.at[pl.ds(base, ROWS_PT)], acc_sh.at[pl.ds(base, ROWS_PT)])
    plsc.subcore_barrier()

    k = CPT_DEG // 4

    @pl.loop(0, 4)
    def _(p):
        @pl.loop(0, k)
        def _(j):
            pltpu.async_copy(ones_v, acc_sh.at[idx_v.at[p * k + j]], sem, add=True)

        @pl.loop(0, k)
        def _(j):
            pltpu.make_async_copy(ones_v, acc_sh.at[idx_v.at[p * k + j]], sem).wait()

    plsc.subcore_barrier()
    pltpu.sync_copy(acc_sh.at[pl.ds(base, ROWS_PT)], out_hbm.at[c, pl.ds(base, ROWS_PT)])


@functools.partial(
    pl.kernel,
    mesh=_mesh,
    out_type=jax.ShapeDtypeStruct((NSC, NP, DH), jnp.float32),
    scratch_types=[
        pltpu.VMEM((CPT, CH), jnp.int32),
        pltpu.VMEM((CPT, CH), jnp.int32),
    ]
    + [pltpu.VMEM((CH, DH), jnp.float32)] * NBUF
    + [pltpu.VMEM_SHARED((NP, DH), jnp.float32)]
    + [pltpu.SemaphoreType.DMA] * (2 * NBUF),
    compiler_params=pltpu.CompilerParams(use_tc_tiling_on_sc=False),
)
def _scatter_kernel(hs_hbm, edge_hbm, out_hbm, idx_s, idx_d,
                    b0, b1, b2, b3, acc_sh, s0, s1, s2, s3, t0, t1, t2, t3):
    c = lax.axis_index("c")
    s = lax.axis_index("s")
    base = s * ROWS_PT
    pltpu.sync_copy(edge_hbm.at[0, pl.ds(s * CPT, CPT)], idx_s)
    pltpu.sync_copy(edge_hbm.at[1, pl.ds(s * CPT, CPT)], idx_d)
    pltpu.sync_copy(hs_hbm.at[c, pl.ds(base, ROWS_PT)],
                    acc_sh.at[pl.ds(base, ROWS_PT)])
    plsc.subcore_barrier()

    bufs = (b0, b1, b2, b3)
    gsem = (s0, s1, s2, s3)
    ssem = (t0, t1, t2, t3)
    tbl = hs_hbm.at[c]

    for k in range(NBUF - 1):
        pltpu.async_copy(tbl.at[idx_s.at[k]], bufs[k], gsem[k])

    @pl.loop(0, CPT // NBUF)
    def _(i):
        j = i * NBUF
        for k in range(NBUF):
            nxt = j + k + NBUF - 1
            b = (k + NBUF - 1) % NBUF

            @pl.when(nxt < CPT)
            def _():
                @pl.when(nxt >= NBUF)
                def _():
                    pltpu.make_async_copy(bufs[b], acc_sh.at[idx_d.at[nxt]],
                                          ssem[b]).wait()

                pltpu.async_copy(tbl.at[idx_s.at[nxt]], bufs[b], gsem[b])

            pltpu.make_async_copy(tbl.at[idx_s.at[j + k]], bufs[k], gsem[k]).wait()
            pltpu.async_copy(bufs[k], acc_sh.at[idx_d.at[j + k]], ssem[k], add=True)

    for k in range(NBUF):
        pltpu.make_async_copy(bufs[k], acc_sh.at[idx_d.at[k]], ssem[k]).wait()

    plsc.subcore_barrier()
    pltpu.sync_copy(acc_sh.at[pl.ds(base, ROWS_PT)],
                    out_hbm.at[c, pl.ds(base, ROWS_PT)])


def _dis(deg_arr):
    deg = deg_arr[0, :, 0] + deg_arr[1, :, 0] + 1.0
    return lax.rsqrt(deg)


def _split_store(out_ref, h):
    out_ref[0, :, :] = h[:, :DH]
    out_ref[1, :, :] = h[:, DH:]


def _unsplit(acc_ref):
    return jnp.concatenate([acc_ref[0, :, :], acc_ref[1, :, :]], axis=1)


def _tc0_body(x_ref, w_ref, out_ref):
    out_ref[...] = jnp.dot(x_ref[...], w_ref[...],
                           preferred_element_type=jnp.float32)


def _tc1_body(h_ref, deg_ref, out_ref):
    d = _dis(deg_ref[...])
    _split_store(out_ref, h_ref[...] * d[:, None])


def _tc2_body(s1_ref, deg_ref, b_ref, w_ref, out_ref):
    d = _dis(deg_ref[...])
    h = jnp.maximum(_unsplit(s1_ref) * d[:, None] + b_ref[...], 0.0)
    h = jnp.dot(h, w_ref[...], preferred_element_type=jnp.float32) * d[:, None]
    _split_store(out_ref, h)


def _tc3_body(s2_ref, deg_ref, b_ref, wr_ref, br_ref, out_ref):
    d = _dis(deg_ref[...])
    h = jnp.maximum(_unsplit(s2_ref) * d[:, None] + b_ref[...], 0.0)
    out_ref[...] = jnp.dot(h, wr_ref[...], preferred_element_type=jnp.float32) + br_ref[...]


_split_shape = jax.ShapeDtypeStruct((NSC, NP, DH), jnp.float32)
_tc0 = pl.pallas_call(_tc0_body, out_shape=jax.ShapeDtypeStruct((NP, D), jnp.float32))
_tc1 = pl.pallas_call(_tc1_body, out_shape=_split_shape)
_tc2 = pl.pallas_call(_tc2_body, out_shape=_split_shape)
_tc3 = pl.pallas_call(_tc3_body, out_shape=jax.ShapeDtypeStruct((NP, 1), jnp.float32))


def kernel(x, edge_index, W1, b1, W2, b2, Wr, br):
    ei = edge_index.astype(jnp.int32)
    edges = jnp.pad(ei, ((0, 0), (0, EP - E)),
                    constant_values=N).reshape(2, EP // CH, CH)
    xp = jnp.pad(x, ((0, NP - N), (0, 0)))
    ones16 = jnp.ones((CH, 16), jnp.float32)
    zeros_np = jnp.zeros((NP, 16), jnp.float32)

    h1 = _tc0(xp, W1)
    deg_a = _deg_kernel(edges, ones16, zeros_np)
    hs1 = _tc1(h1, deg_a)
    acc1 = _scatter_kernel(hs1, edges)
    hs2 = _tc2(acc1, deg_a, b1.reshape(1, D), W2)
    acc2 = _scatter_kernel(hs2, edges)
    y = _tc3(acc2, deg_a, b2.reshape(1, D), Wr, br.reshape(1, 1))
    return y[:N]


# speedup vs baseline: 1.0805x; 1.0805x over previous
import functools

import jax
import jax.numpy as jnp
from jax import lax
from jax.experimental import pallas as pl
from jax.experimental.pallas import tpu as pltpu
from jax.experimental.pallas import tpu_sc as plsc

N = 10000
NP = 10240
E = 320000
D = 128
DH = 64
CH = 128
NSC = 2
NSUB = 16
EP = 327680
CPT = EP // NSUB // CH
CPT_DEG = EP // (NSC * NSUB) // CH
NBUF = 4
ROWS_PT = NP // NSUB

_mesh = plsc.VectorSubcoreMesh(core_axis_name="c", subcore_axis_name="s")


@functools.partial(
    pl.kernel,
    mesh=_mesh,
    out_type=jax.ShapeDtypeStruct((NSC, NP, 16), jnp.float32),
    scratch_types=[
        pltpu.VMEM((CPT_DEG, CH), jnp.int32),
        pltpu.VMEM((CH, 16), jnp.float32),
        pltpu.VMEM_SHARED((NP, 16), jnp.float32),
        pltpu.SemaphoreType.DMA,
    ],
    compiler_params=pltpu.CompilerParams(use_tc_tiling_on_sc=False),
)
def _deg_kernel(dst_hbm, ones_hbm, zeros_hbm, out_hbm, idx_v, ones_v, acc_sh, sem):
    c = lax.axis_index("c")
    s = lax.axis_index("s")
    w = c * NSUB + s
    base = s * ROWS_PT
    pltpu.sync_copy(dst_hbm.at[w], idx_v)
    pltpu.sync_copy(ones_hbm, ones_v)
    pltpu.sync_copy(zeros_hbm.at[pl.ds(base, ROWS_PT)], acc_sh.at[pl.ds(base, ROWS_PT)])
    plsc.subcore_barrier()

    k = CPT_DEG // 4

    @pl.loop(0, 4)
    def _(p):
        @pl.loop(0, k)
        def _(j):
            pltpu.async_copy(ones_v, acc_sh.at[idx_v.at[p * k + j]], sem, add=True)

        @pl.loop(0, k)
        def _(j):
            pltpu.make_async_copy(ones_v, acc_sh.at[idx_v.at[p * k + j]], sem).wait()

    plsc.subcore_barrier()
    pltpu.sync_copy(acc_sh.at[pl.ds(base, ROWS_PT)], out_hbm.at[c, pl.ds(base, ROWS_PT)])


@functools.partial(
    pl.kernel,
    mesh=_mesh,
    out_type=jax.ShapeDtypeStruct((NSC, NP, DH), jnp.float32),
    scratch_types=[
        pltpu.VMEM((CPT, CH), jnp.int32),
        pltpu.VMEM((CPT, CH), jnp.int32),
    ]
    + [pltpu.VMEM((CH, DH), jnp.float32)] * NBUF
    + [pltpu.VMEM_SHARED((NP, DH), jnp.float32)]
    + [pltpu.SemaphoreType.DMA] * (2 * NBUF),
    compiler_params=pltpu.CompilerParams(use_tc_tiling_on_sc=False),
)
def _scatter_kernel(hs_hbm, src_hbm, dst_hbm, out_hbm, idx_s, idx_d,
                    b0, b1, b2, b3, acc_sh, s0, s1, s2, s3, t0, t1, t2, t3):
    c = lax.axis_index("c")
    s = lax.axis_index("s")
    base = s * ROWS_PT
    pltpu.sync_copy(src_hbm.at[s], idx_s)
    pltpu.sync_copy(dst_hbm.at[s], idx_d)
    pltpu.sync_copy(hs_hbm.at[c, pl.ds(base, ROWS_PT)],
                    acc_sh.at[pl.ds(base, ROWS_PT)])
    plsc.subcore_barrier()

    bufs = (b0, b1, b2, b3)
    gsem = (s0, s1, s2, s3)
    ssem = (t0, t1, t2, t3)
    tbl = hs_hbm.at[c]

    for k in range(NBUF - 1):
        pltpu.async_copy(tbl.at[idx_s.at[k]], bufs[k], gsem[k])

    @pl.loop(0, CPT // NBUF)
    def _(i):
        j = i * NBUF
        for k in range(NBUF):
            nxt = j + k + NBUF - 1
            b = (k + NBUF - 1) % NBUF

            @pl.when(nxt < CPT)
            def _():
                @pl.when(nxt >= NBUF)
                def _():
                    pltpu.make_async_copy(bufs[b], acc_sh.at[idx_d.at[nxt]],
                                          ssem[b]).wait()

                pltpu.async_copy(tbl.at[idx_s.at[nxt]], bufs[b], gsem[b])

            pltpu.make_async_copy(tbl.at[idx_s.at[j + k]], bufs[k], gsem[k]).wait()
            pltpu.async_copy(bufs[k], acc_sh.at[idx_d.at[j + k]], ssem[k], add=True)

    for k in range(NBUF):
        pltpu.make_async_copy(bufs[k], acc_sh.at[idx_d.at[k]], ssem[k]).wait()

    plsc.subcore_barrier()
    pltpu.sync_copy(acc_sh.at[pl.ds(base, ROWS_PT)],
                    out_hbm.at[c, pl.ds(base, ROWS_PT)])


def _dis(deg_arr):
    deg = deg_arr[0, :, 0] + deg_arr[1, :, 0] + 1.0
    return lax.rsqrt(deg)


def _split_store(out_ref, h):
    out_ref[0, :, :] = h[:, :DH]
    out_ref[1, :, :] = h[:, DH:]


def _unsplit(acc_ref):
    return jnp.concatenate([acc_ref[0, :, :], acc_ref[1, :, :]], axis=1)


def _tc0_body(x_ref, w_ref, out_ref):
    out_ref[...] = jnp.dot(x_ref[...], w_ref[...],
                           preferred_element_type=jnp.float32)


def _tc1_body(h_ref, deg_ref, out_ref):
    d = _dis(deg_ref[...])
    _split_store(out_ref, h_ref[...] * d[:, None])


def _tc2_body(s1_ref, deg_ref, b_ref, w_ref, out_ref):
    d = _dis(deg_ref[...])
    h = jnp.maximum(_unsplit(s1_ref) * d[:, None] + b_ref[...], 0.0)
    h = jnp.dot(h, w_ref[...], preferred_element_type=jnp.float32) * d[:, None]
    _split_store(out_ref, h)


def _tc3_body(s2_ref, deg_ref, b_ref, wr_ref, br_ref, out_ref):
    d = _dis(deg_ref[...])
    h = jnp.maximum(_unsplit(s2_ref) * d[:, None] + b_ref[...], 0.0)
    out_ref[...] = jnp.dot(h, wr_ref[...], preferred_element_type=jnp.float32) + br_ref[...]


_split_shape = jax.ShapeDtypeStruct((NSC, NP, DH), jnp.float32)
_tc0 = pl.pallas_call(_tc0_body, out_shape=jax.ShapeDtypeStruct((NP, D), jnp.float32))
_tc1 = pl.pallas_call(_tc1_body, out_shape=_split_shape)
_tc2 = pl.pallas_call(_tc2_body, out_shape=_split_shape)
_tc3 = pl.pallas_call(_tc3_body, out_shape=jax.ShapeDtypeStruct((NP, 1), jnp.float32))


def kernel(x, edge_index, W1, b1, W2, b2, Wr, br):
    src = edge_index[0].astype(jnp.int32)
    dst = edge_index[1].astype(jnp.int32)
    pad = EP - E
    sentinel = jnp.full((pad,), N, jnp.int32)
    src_p = jnp.concatenate([src, sentinel])
    dst_p = jnp.concatenate([dst, sentinel])
    src16 = src_p.reshape(NSUB, CPT, CH)
    dst16 = dst_p.reshape(NSUB, CPT, CH)
    dst32 = dst_p.reshape(NSC * NSUB, CPT_DEG, CH)
    xp = jnp.pad(x, ((0, NP - N), (0, 0)))
    ones16 = jnp.ones((CH, 16), jnp.float32)
    zeros_np = jnp.zeros((NP, 16), jnp.float32)

    h1 = _tc0(xp, W1)
    deg_a = _deg_kernel(dst32, ones16, zeros_np)
    hs1 = _tc1(h1, deg_a)
    acc1 = _scatter_kernel(hs1, src16, dst16)
    hs2 = _tc2(acc1, deg_a, b1.reshape(1, D), W2)
    acc2 = _scatter_kernel(hs2, src16, dst16)
    y = _tc3(acc2, deg_a, b2.reshape(1, D), Wr, br.reshape(1, 1))
    return y[:N]


# speedup vs baseline: 1.2175x; 1.1269x over previous
import functools

import jax
import jax.numpy as jnp
from jax import lax
from jax.experimental import pallas as pl
from jax.experimental.pallas import tpu as pltpu
from jax.experimental.pallas import tpu_sc as plsc

N = 10000
NP = 10240
E = 320000
D = 128
DH = 64
CH = 128
NSC = 2
NSUB = 16
EP = 327680
CPT = EP // NSUB // CH
CPT_DEG = EP // (NSC * NSUB) // CH
NBUF = 5
ROWS_PT = NP // NSUB

_mesh = plsc.VectorSubcoreMesh(core_axis_name="c", subcore_axis_name="s")


@functools.partial(
    pl.kernel,
    mesh=_mesh,
    out_type=jax.ShapeDtypeStruct((NSC, NP, 16), jnp.float32),
    scratch_types=[
        pltpu.VMEM((CPT_DEG, CH), jnp.int32),
        pltpu.VMEM((CH, 16), jnp.float32),
        pltpu.VMEM_SHARED((NP, 16), jnp.float32),
        pltpu.SemaphoreType.DMA,
    ],
    compiler_params=pltpu.CompilerParams(use_tc_tiling_on_sc=False),
)
def _deg_kernel(dst_hbm, ones_hbm, zeros_hbm, out_hbm, idx_v, ones_v, acc_sh, sem):
    c = lax.axis_index("c")
    s = lax.axis_index("s")
    w = c * NSUB + s
    base = s * ROWS_PT
    pltpu.sync_copy(dst_hbm.at[w], idx_v)
    pltpu.sync_copy(ones_hbm, ones_v)
    pltpu.sync_copy(zeros_hbm.at[pl.ds(base, ROWS_PT)], acc_sh.at[pl.ds(base, ROWS_PT)])
    plsc.subcore_barrier()

    k = CPT_DEG // 4

    @pl.loop(0, 4)
    def _(p):
        @pl.loop(0, k)
        def _(j):
            pltpu.async_copy(ones_v, acc_sh.at[idx_v.at[p * k + j]], sem, add=True)

        @pl.loop(0, k)
        def _(j):
            pltpu.make_async_copy(ones_v, acc_sh.at[idx_v.at[p * k + j]], sem).wait()

    plsc.subcore_barrier()
    pltpu.sync_copy(acc_sh.at[pl.ds(base, ROWS_PT)], out_hbm.at[c, pl.ds(base, ROWS_PT)])


@functools.partial(
    pl.kernel,
    mesh=_mesh,
    out_type=jax.ShapeDtypeStruct((NSC, NP, DH), jnp.float32),
    scratch_types=[
        pltpu.VMEM((CPT, CH), jnp.int32),
        pltpu.VMEM((CPT, CH), jnp.int32),
    ]
    + [pltpu.VMEM((CH, DH), jnp.float32)] * NBUF
    + [pltpu.VMEM_SHARED((NP, DH), jnp.float32)]
    + [pltpu.SemaphoreType.DMA] * (2 * NBUF),
    compiler_params=pltpu.CompilerParams(use_tc_tiling_on_sc=False),
)
def _scatter_kernel(hs_hbm, src_hbm, dst_hbm, out_hbm, idx_s, idx_d,
                    b0, b1, b2, b3, b4, acc_sh, s0, s1, s2, s3, s4,
                    t0, t1, t2, t3, t4):
    c = lax.axis_index("c")
    s = lax.axis_index("s")
    base = s * ROWS_PT
    pltpu.sync_copy(src_hbm.at[s], idx_s)
    pltpu.sync_copy(dst_hbm.at[s], idx_d)
    pltpu.sync_copy(hs_hbm.at[c, pl.ds(base, ROWS_PT)],
                    acc_sh.at[pl.ds(base, ROWS_PT)])
    plsc.subcore_barrier()

    bufs = (b0, b1, b2, b3, b4)
    gsem = (s0, s1, s2, s3, s4)
    ssem = (t0, t1, t2, t3, t4)
    tbl = hs_hbm.at[c]

    for k in range(NBUF - 1):
        pltpu.async_copy(tbl.at[idx_s.at[k]], bufs[k], gsem[k])

    @pl.loop(0, CPT // NBUF)
    def _(i):
        j = i * NBUF
        for k in range(NBUF):
            nxt = j + k + NBUF - 1
            b = (k + NBUF - 1) % NBUF

            @pl.when(nxt < CPT)
            def _():
                @pl.when(nxt >= NBUF)
                def _():
                    pltpu.make_async_copy(bufs[b], acc_sh.at[idx_d.at[nxt]],
                                          ssem[b]).wait()

                pltpu.async_copy(tbl.at[idx_s.at[nxt]], bufs[b], gsem[b])

            pltpu.make_async_copy(tbl.at[idx_s.at[j + k]], bufs[k], gsem[k]).wait()
            pltpu.async_copy(bufs[k], acc_sh.at[idx_d.at[j + k]], ssem[k], add=True)

    for k in range(NBUF):
        pltpu.make_async_copy(bufs[k], acc_sh.at[idx_d.at[k]], ssem[k]).wait()

    plsc.subcore_barrier()
    pltpu.sync_copy(acc_sh.at[pl.ds(base, ROWS_PT)],
                    out_hbm.at[c, pl.ds(base, ROWS_PT)])


def _dis(deg_arr):
    deg = deg_arr[0, :, 0] + deg_arr[1, :, 0] + 1.0
    return lax.rsqrt(deg)


def _split_store(out_ref, h):
    out_ref[0, :, :] = h[:, :DH]
    out_ref[1, :, :] = h[:, DH:]


def _unsplit(acc_ref):
    return jnp.concatenate([acc_ref[0, :, :], acc_ref[1, :, :]], axis=1)


def _tc1_body(x_ref, w_ref, deg_ref, out_ref):
    d = _dis(deg_ref[...])
    h = jnp.dot(x_ref[...], w_ref[...], preferred_element_type=jnp.float32)
    _split_store(out_ref, h * d[:, None])


def _tc2_body(s1_ref, deg_ref, b_ref, w_ref, out_ref):
    d = _dis(deg_ref[...])
    h = jnp.maximum(_unsplit(s1_ref) * d[:, None] + b_ref[...], 0.0)
    h = jnp.dot(h, w_ref[...], preferred_element_type=jnp.float32) * d[:, None]
    _split_store(out_ref, h)


def _tc3_body(s2_ref, deg_ref, b_ref, wr_ref, br_ref, out_ref):
    d = _dis(deg_ref[...])
    h = jnp.maximum(_unsplit(s2_ref) * d[:, None] + b_ref[...], 0.0)
    out_ref[...] = jnp.dot(h, wr_ref[...], preferred_element_type=jnp.float32) + br_ref[...]


_split_shape = jax.ShapeDtypeStruct((NSC, NP, DH), jnp.float32)
_tc1 = pl.pallas_call(_tc1_body, out_shape=_split_shape)
_tc2 = pl.pallas_call(_tc2_body, out_shape=_split_shape)
_tc3 = pl.pallas_call(_tc3_body, out_shape=jax.ShapeDtypeStruct((NP, 1), jnp.float32))


def kernel(x, edge_index, W1, b1, W2, b2, Wr, br):
    src = edge_index[0].astype(jnp.int32)
    dst = edge_index[1].astype(jnp.int32)
    pad = EP - E
    sentinel = jnp.full((pad,), N, jnp.int32)
    src_p = jnp.concatenate([src, sentinel])
    dst_p = jnp.concatenate([dst, sentinel])
    src16 = src_p.reshape(NSUB, CPT, CH)
    dst16 = dst_p.reshape(NSUB, CPT, CH)
    dst32 = dst_p.reshape(NSC * NSUB, CPT_DEG, CH)
    xp = jnp.pad(x, ((0, NP - N), (0, 0)))
    ones16 = jnp.ones((CH, 16), jnp.float32)
    zeros_np = jnp.zeros((NP, 16), jnp.float32)

    deg_a = _deg_kernel(dst32, ones16, zeros_np)
    hs1 = _tc1(xp, W1, deg_a)
    acc1 = _scatter_kernel(hs1, src16, dst16)
    hs2 = _tc2(acc1, deg_a, b1.reshape(1, D), W2)
    acc2 = _scatter_kernel(hs2, src16, dst16)
    y = _tc3(acc2, deg_a, b2.reshape(1, D), Wr, br.reshape(1, 1))
    return y[:N]
